# baseline (speedup 1.0000x reference)
import functools

import jax
import jax.numpy as jnp
from jax import lax
from jax.experimental import pallas as pl
from jax.experimental.pallas import tpu as pltpu
from jax.experimental.pallas import tpu_sc as plsc

N_NODES = 10000
N_PAD = 10240
N_EDGES = 320000
D_IN = 128
H1, H2, GCN_OUT = 32, 64, 64
N_GRAPHS = 64
N_CLASSES = 10

F = 64
PF = 2 * F
ONES_COL = 32

NC, NS = 2, 16
NW = NC * NS
CHUNK = 125
CPT = N_EDGES // NW // CHUNK
ROWS_PER_TILE = N_PAD // NS

NBUF = 8
PREF = 4


mesh = plsc.VectorSubcoreMesh(core_axis_name="c", subcore_axis_name="s")


@functools.partial(
    pl.kernel,
    out_type=jax.ShapeDtypeStruct((N_PAD, PF), jnp.bfloat16),
    mesh=mesh,
    compiler_params=pltpu.CompilerParams(use_tc_tiling_on_sc=False),
    scratch_types=[
        pltpu.VMEM((CPT, CHUNK), jnp.int32),
        pltpu.VMEM((CPT, CHUNK), jnp.int32),
        pltpu.VMEM((NBUF, CHUNK, F), jnp.bfloat16),
        pltpu.VMEM_SHARED((N_PAD, F), jnp.bfloat16),
        pltpu.SemaphoreType.DMA,
        [pltpu.SemaphoreType.DMA] * NBUF,
        [pltpu.SemaphoreType.DMA] * NBUF,
    ],
)
def _agg(hw_hbm, edges_hbm, out_hbm,
         src_v, dst_v, rows_v, acc_sh, psem, gsem, ssem):
  c = lax.axis_index("c")
  s = lax.axis_index("s")
  wid = c * NS + s
  row0 = s * ROWS_PER_TILE
  pltpu.async_copy(edges_hbm.at[0].at[pl.ds(wid * CPT, CPT)], src_v, psem)
  pltpu.async_copy(edges_hbm.at[1].at[pl.ds(wid * CPT, CPT)], dst_v, psem)
  pltpu.make_async_copy(edges_hbm.at[0].at[pl.ds(wid * CPT, CPT)], src_v,
                        psem).wait()
  pltpu.make_async_copy(edges_hbm.at[1].at[pl.ds(wid * CPT, CPT)], dst_v,
                        psem).wait()
  zvec = jnp.zeros((32,), jnp.bfloat16)

  def zbody(r, carry):
    rows_v[0, r, pl.ds(0, 32)] = zvec
    rows_v[0, r, pl.ds(32, 32)] = zvec
    return carry

  lax.fori_loop(0, CHUNK, zbody, 0)
  for q in range(ROWS_PER_TILE // CHUNK):
    pltpu.sync_copy(rows_v.at[0], acc_sh.at[pl.ds(row0 + q * CHUNK, CHUNK)])
  rem = ROWS_PER_TILE % CHUNK
  pltpu.sync_copy(rows_v.at[0].at[pl.ds(0, rem)],
                  acc_sh.at[pl.ds(row0 + (ROWS_PER_TILE // CHUNK) * CHUNK,
                                  rem)])
  plsc.subcore_barrier()

  def g_start(j, b):
    pltpu.async_copy(hw_hbm.at[src_v.at[j]], rows_v.at[b], gsem[b])

  def g_wait(j, b):
    pltpu.make_async_copy(hw_hbm.at[src_v.at[j]], rows_v.at[b],
                          gsem[b]).wait()

  def s_start(j, b):
    pltpu.async_copy(rows_v.at[b], acc_sh.at[dst_v.at[j]], ssem[b],
                     add=True)

  def s_wait(j, b):
    pltpu.make_async_copy(rows_v.at[b], acc_sh.at[dst_v.at[j]],
                          ssem[b]).wait()

  for j in range(PREF):
    g_start(j, j)

  def body(i, carry):
    for k in range(NBUF):
      j = NBUF * i + k
      jg = j + PREF
      bg = (k + PREF) % NBUF
      @pl.when(jg < CPT)
      def _():
        @pl.when(jg >= NBUF)
        def _():
          s_wait(jg - NBUF, bg)
        g_start(jg, bg)
      g_wait(j, k)
      s_start(j, k)
    return carry

  lax.fori_loop(0, CPT // NBUF, body, 0)
  for k in range(NBUF):
    s_wait(CPT - NBUF + k, k)
  plsc.subcore_barrier()
  pltpu.sync_copy(acc_sh.at[pl.ds(row0, ROWS_PER_TILE)],
                  out_hbm.at[pl.ds(row0, ROWS_PER_TILE), pl.ds(c * F, F)])


_BLK = 2048
HALF = N_PAD // 2
_PBLK = 1024


def _mm_ones_body(xl_ref, xr_ref, w_ref, o_ref):
  hl = jnp.dot(xl_ref[...], w_ref[...], preferred_element_type=jnp.float32)
  hr = jnp.dot(xr_ref[...], w_ref[...], preferred_element_type=jnp.float32)
  col = lax.broadcasted_iota(jnp.int32, (_PBLK, F), 1)
  ones = (col == ONES_COL).astype(jnp.float32)
  o_ref[...] = jnp.concatenate([hl + ones, hr + ones],
                               axis=1).astype(jnp.bfloat16)


def _dinv(p1, sum2_ref, sel2_ref):
  deg = jnp.dot(p1, sel2_ref[...], preferred_element_type=jnp.float32)
  return 1.0 / jnp.maximum(deg, 1.0)


def _norm_mm_body(pl_ref, pr_ref, p1l_ref, p1r_ref, sum2_ref, sel2_ref,
                  w_ref, o_ref):
  def half(p_ref, p1_ref):
    s = jnp.dot(p_ref[...], sum2_ref[...], preferred_element_type=jnp.float32)
    h = jnp.maximum(s * _dinv(p1_ref[...], sum2_ref, sel2_ref), 0.0)
    return jnp.dot(h, w_ref[...], preferred_element_type=jnp.float32)

  o_ref[...] = jnp.concatenate([half(pl_ref, p1l_ref),
                                half(pr_ref, p1r_ref)],
                               axis=1).astype(jnp.bfloat16)


def _pool_body(p_ref, p1_ref, sum2_ref, sel2_ref, pids_ref, wd_ref, bd_ref,
               o_ref):
  s = jnp.dot(p_ref[...], sum2_ref[...], preferred_element_type=jnp.float32)
  reprs = jnp.maximum(s * _dinv(p1_ref[...], sum2_ref, sel2_ref), 0.0)
  gids = lax.broadcasted_iota(jnp.int32, (N_PAD, N_GRAPHS), 1)
  onehot = (pids_ref[...] == gids).astype(jnp.float32)
  pooled = lax.dot_general(onehot, reprs, (((0,), (0,)), ((), ())),
                           preferred_element_type=jnp.float32)
  o_ref[...] = jnp.dot(pooled, wd_ref[...],
                       preferred_element_type=jnp.float32) + bd_ref[...]


def kernel(x, edge_index, pool_ids, W1, W2, W3, Wd, bd):
  f32 = jnp.float32
  is_src = lax.broadcasted_iota(jnp.int32, (2, N_EDGES), 0) == 0
  perm = jnp.where(edge_index < N_PAD // 2, 2 * edge_index,
                   2 * edge_index - (N_PAD - 1))
  edges = jnp.where(is_src, perm, edge_index).reshape(2, NW * CPT, CHUNK)

  W1pad = jnp.concatenate([W1, jnp.zeros((D_IN, F - H1), f32)], axis=1)
  W2pad = jnp.concatenate([W2, jnp.zeros((F - H1, H2), f32)], axis=0)
  r2 = lax.broadcasted_iota(jnp.int32, (PF, F), 0)
  c2 = lax.broadcasted_iota(jnp.int32, (PF, F), 1)
  sum2 = (r2 % F == c2).astype(f32)
  r1 = lax.broadcasted_iota(jnp.int32, (PF, 1), 0)
  sel2 = ((r1 % F) == ONES_COL).astype(f32)


  grid10 = (N_PAD // _BLK,)
  row_blk = lambda w: pl.BlockSpec((_BLK, w), lambda i: (i, 0))
  full = lambda a: pl.BlockSpec(a.shape, lambda i: (0,) * a.ndim)

  nb = HALF // _PBLK
  hw1 = pl.pallas_call(
      _mm_ones_body,
      grid=(nb,),
      in_specs=[pl.BlockSpec((_PBLK, D_IN), lambda i: (i, 0)),
                pl.BlockSpec((_PBLK, D_IN), lambda i: (i + nb, 0)),
                full(W1pad)],
      out_specs=pl.BlockSpec((_PBLK, PF), lambda i: (i, 0)),
      out_shape=jax.ShapeDtypeStruct((HALF, PF), jnp.bfloat16),
  )(x, x, W1pad).reshape(N_PAD, F)

  p1 = _agg(hw1, edges)

  pk_blk = lambda off: pl.BlockSpec((_PBLK, PF), lambda i: (i + off, 0))
  hw2 = pl.pallas_call(
      _norm_mm_body,
      grid=(nb,),
      in_specs=[pk_blk(0), pk_blk(nb), pk_blk(0), pk_blk(nb),
                full(sum2), full(sel2), full(W2pad)],
      out_specs=pl.BlockSpec((_PBLK, PF), lambda i: (i, 0)),
      out_shape=jax.ShapeDtypeStruct((HALF, PF), jnp.bfloat16),
  )(p1, p1, p1, p1, sum2, sel2, W2pad).reshape(N_PAD, F)

  p2 = _agg(hw2, edges)

  hw3 = pl.pallas_call(
      _norm_mm_body,
      grid=(nb,),
      in_specs=[pk_blk(0), pk_blk(nb), pk_blk(0), pk_blk(nb),
                full(sum2), full(sel2), full(W3)],
      out_specs=pl.BlockSpec((_PBLK, PF), lambda i: (i, 0)),
      out_shape=jax.ShapeDtypeStruct((HALF, PF), jnp.bfloat16),
  )(p2, p2, p1, p1, sum2, sel2, W3).reshape(N_PAD, F)

  p3 = _agg(hw3, edges)

  blk0 = lambda a: pl.BlockSpec(a.shape, lambda: (0,) * a.ndim)
  pids2d = jnp.concatenate(
      [pool_ids, jnp.full((N_PAD - N_NODES,), N_GRAPHS, jnp.int32)]
  ).reshape(N_PAD, 1)
  bd2d = bd.reshape(1, N_CLASSES)
  out = pl.pallas_call(
      _pool_body,
      in_specs=[blk0(p3), blk0(p1), blk0(sum2), blk0(sel2), blk0(pids2d),
                blk0(Wd), blk0(bd2d)],
      out_specs=pl.BlockSpec((N_GRAPHS, N_CLASSES), lambda: (0, 0)),
      out_shape=jax.ShapeDtypeStruct((N_GRAPHS, N_CLASSES), f32),
  )(p3, p1, sum2, sel2, pids2d, Wd, bd2d)

  return out


# speedup vs baseline: 1.0062x; 1.0062x over previous
import functools

import jax
import jax.numpy as jnp
from jax import lax
from jax.experimental import pallas as pl
from jax.experimental.pallas import tpu as pltpu
from jax.experimental.pallas import tpu_sc as plsc

N_NODES = 10000
N_PAD = 10240
N_EDGES = 320000
D_IN = 128
H1, H2, GCN_OUT = 32, 64, 64
N_GRAPHS = 64
N_CLASSES = 10

F = 64
PF = 2 * F
ONES_COL = 32

NC, NS = 2, 16
NW = NC * NS
CHUNK = 125
CPT = N_EDGES // NW // CHUNK
ROWS_PER_TILE = N_PAD // NS

NBUF = 8
PREF = 4


mesh = plsc.VectorSubcoreMesh(core_axis_name="c", subcore_axis_name="s")


@functools.partial(
    pl.kernel,
    out_type=jax.ShapeDtypeStruct((N_PAD, PF), jnp.bfloat16),
    mesh=mesh,
    compiler_params=pltpu.CompilerParams(use_tc_tiling_on_sc=False),
    scratch_types=[
        pltpu.VMEM((CPT, CHUNK), jnp.int32),
        pltpu.VMEM((CPT, CHUNK), jnp.int32),
        pltpu.VMEM((NBUF, CHUNK, F), jnp.bfloat16),
        pltpu.VMEM_SHARED((N_PAD, F), jnp.bfloat16),
        pltpu.SemaphoreType.DMA,
        [pltpu.SemaphoreType.DMA] * NBUF,
        [pltpu.SemaphoreType.DMA] * NBUF,
    ],
)
def _agg(hw_hbm, edges_hbm, zero_hbm, out_hbm,
         src_v, dst_v, rows_v, acc_sh, psem, gsem, ssem):
  c = lax.axis_index("c")
  s = lax.axis_index("s")
  wid = c * NS + s
  row0 = s * ROWS_PER_TILE
  pltpu.async_copy(edges_hbm.at[0].at[pl.ds(wid * CPT, CPT)], src_v, psem)
  pltpu.async_copy(edges_hbm.at[1].at[pl.ds(wid * CPT, CPT)], dst_v, psem)
  pltpu.async_copy(zero_hbm.at[pl.ds(row0, ROWS_PER_TILE)],
                   acc_sh.at[pl.ds(row0, ROWS_PER_TILE)], psem)
  pltpu.make_async_copy(edges_hbm.at[0].at[pl.ds(wid * CPT, CPT)], src_v,
                        psem).wait()
  pltpu.make_async_copy(edges_hbm.at[1].at[pl.ds(wid * CPT, CPT)], dst_v,
                        psem).wait()
  pltpu.make_async_copy(zero_hbm.at[pl.ds(row0, ROWS_PER_TILE)],
                        acc_sh.at[pl.ds(row0, ROWS_PER_TILE)], psem).wait()
  plsc.subcore_barrier()

  def g_start(j, b):
    pltpu.async_copy(hw_hbm.at[src_v.at[j]], rows_v.at[b], gsem[b])

  def g_wait(j, b):
    pltpu.make_async_copy(hw_hbm.at[src_v.at[j]], rows_v.at[b],
                          gsem[b]).wait()

  def s_start(j, b):
    pltpu.async_copy(rows_v.at[b], acc_sh.at[dst_v.at[j]], ssem[b],
                     add=True)

  def s_wait(j, b):
    pltpu.make_async_copy(rows_v.at[b], acc_sh.at[dst_v.at[j]],
                          ssem[b]).wait()

  for j in range(PREF):
    g_start(j, j)

  def body(i, carry):
    for k in range(NBUF):
      j = NBUF * i + k
      jg = j + PREF
      bg = (k + PREF) % NBUF
      @pl.when(jg < CPT)
      def _():
        @pl.when(jg >= NBUF)
        def _():
          s_wait(jg - NBUF, bg)
        g_start(jg, bg)
      g_wait(j, k)
      s_start(j, k)
    return carry

  lax.fori_loop(0, CPT // NBUF, body, 0)
  for k in range(NBUF):
    s_wait(CPT - NBUF + k, k)
  plsc.subcore_barrier()
  pltpu.sync_copy(acc_sh.at[pl.ds(row0, ROWS_PER_TILE)],
                  out_hbm.at[pl.ds(row0, ROWS_PER_TILE), pl.ds(c * F, F)])


_BLK = 2048
HALF = N_PAD // 2
_PBLK = 1024


def _mm_ones_body(xl_ref, xr_ref, w_ref, o_ref):
  hl = jnp.dot(xl_ref[...], w_ref[...], preferred_element_type=jnp.float32)
  hr = jnp.dot(xr_ref[...], w_ref[...], preferred_element_type=jnp.float32)
  col = lax.broadcasted_iota(jnp.int32, (_PBLK, F), 1)
  ones = (col == ONES_COL).astype(jnp.float32)
  o_ref[...] = jnp.concatenate([hl + ones, hr + ones],
                               axis=1).astype(jnp.bfloat16)


def _dinv(p1, sum2_ref, sel2_ref):
  deg = jnp.dot(p1, sel2_ref[...], preferred_element_type=jnp.float32)
  return 1.0 / jnp.maximum(deg, 1.0)


def _norm_mm_body(pl_ref, pr_ref, p1l_ref, p1r_ref, sum2_ref, sel2_ref,
                  w_ref, o_ref):
  def half(p_ref, p1_ref):
    s = jnp.dot(p_ref[...], sum2_ref[...], preferred_element_type=jnp.float32)
    h = jnp.maximum(s * _dinv(p1_ref[...], sum2_ref, sel2_ref), 0.0)
    return jnp.dot(h, w_ref[...], preferred_element_type=jnp.float32)

  o_ref[...] = jnp.concatenate([half(pl_ref, p1l_ref),
                                half(pr_ref, p1r_ref)],
                               axis=1).astype(jnp.bfloat16)


def _pool_body(p_ref, p1_ref, sum2_ref, sel2_ref, pids_ref, wd_ref, bd_ref,
               o_ref):
  s = jnp.dot(p_ref[...], sum2_ref[...], preferred_element_type=jnp.float32)
  reprs = jnp.maximum(s * _dinv(p1_ref[...], sum2_ref, sel2_ref), 0.0)
  gids = lax.broadcasted_iota(jnp.int32, (N_PAD, N_GRAPHS), 1)
  onehot = (pids_ref[...] == gids).astype(jnp.float32)
  pooled = lax.dot_general(onehot, reprs, (((0,), (0,)), ((), ())),
                           preferred_element_type=jnp.float32)
  o_ref[...] = jnp.dot(pooled, wd_ref[...],
                       preferred_element_type=jnp.float32) + bd_ref[...]


def kernel(x, edge_index, pool_ids, W1, W2, W3, Wd, bd):
  f32 = jnp.float32
  is_src = lax.broadcasted_iota(jnp.int32, (2, N_EDGES), 0) == 0
  perm = jnp.where(edge_index < N_PAD // 2, 2 * edge_index,
                   2 * edge_index - (N_PAD - 1))
  edges = jnp.where(is_src, perm, edge_index).reshape(2, NW * CPT, CHUNK)

  W1pad = jnp.concatenate([W1, jnp.zeros((D_IN, F - H1), f32)], axis=1)
  W2pad = jnp.concatenate([W2, jnp.zeros((F - H1, H2), f32)], axis=0)
  r2 = lax.broadcasted_iota(jnp.int32, (PF, F), 0)
  c2 = lax.broadcasted_iota(jnp.int32, (PF, F), 1)
  sum2 = (r2 % F == c2).astype(f32)
  r1 = lax.broadcasted_iota(jnp.int32, (PF, 1), 0)
  sel2 = ((r1 % F) == ONES_COL).astype(f32)

  zero = jnp.zeros((N_PAD, F), jnp.bfloat16)

  grid10 = (N_PAD // _BLK,)
  row_blk = lambda w: pl.BlockSpec((_BLK, w), lambda i: (i, 0))
  full = lambda a: pl.BlockSpec(a.shape, lambda i: (0,) * a.ndim)

  nb = HALF // _PBLK
  hw1 = pl.pallas_call(
      _mm_ones_body,
      grid=(nb,),
      in_specs=[pl.BlockSpec((_PBLK, D_IN), lambda i: (i, 0)),
                pl.BlockSpec((_PBLK, D_IN), lambda i: (i + nb, 0)),
                full(W1pad)],
      out_specs=pl.BlockSpec((_PBLK, PF), lambda i: (i, 0)),
      out_shape=jax.ShapeDtypeStruct((HALF, PF), jnp.bfloat16),
  )(x, x, W1pad).reshape(N_PAD, F)

  p1 = _agg(hw1, edges, zero)

  pk_blk = lambda off: pl.BlockSpec((_PBLK, PF), lambda i: (i + off, 0))
  hw2 = pl.pallas_call(
      _norm_mm_body,
      grid=(nb,),
      in_specs=[pk_blk(0), pk_blk(nb), pk_blk(0), pk_blk(nb),
                full(sum2), full(sel2), full(W2pad)],
      out_specs=pl.BlockSpec((_PBLK, PF), lambda i: (i, 0)),
      out_shape=jax.ShapeDtypeStruct((HALF, PF), jnp.bfloat16),
  )(p1, p1, p1, p1, sum2, sel2, W2pad).reshape(N_PAD, F)

  p2 = _agg(hw2, edges, zero)

  hw3 = pl.pallas_call(
      _norm_mm_body,
      grid=(nb,),
      in_specs=[pk_blk(0), pk_blk(nb), pk_blk(0), pk_blk(nb),
                full(sum2), full(sel2), full(W3)],
      out_specs=pl.BlockSpec((_PBLK, PF), lambda i: (i, 0)),
      out_shape=jax.ShapeDtypeStruct((HALF, PF), jnp.bfloat16),
  )(p2, p2, p1, p1, sum2, sel2, W3).reshape(N_PAD, F)

  p3 = _agg(hw3, edges, zero)

  blk0 = lambda a: pl.BlockSpec(a.shape, lambda: (0,) * a.ndim)
  pids2d = jnp.concatenate(
      [pool_ids, jnp.full((N_PAD - N_NODES,), N_GRAPHS, jnp.int32)]
  ).reshape(N_PAD, 1)
  bd2d = bd.reshape(1, N_CLASSES)
  out = pl.pallas_call(
      _pool_body,
      in_specs=[blk0(p3), blk0(p1), blk0(sum2), blk0(sel2), blk0(pids2d),
                blk0(Wd), blk0(bd2d)],
      out_specs=pl.BlockSpec((N_GRAPHS, N_CLASSES), lambda: (0, 0)),
      out_shape=jax.ShapeDtypeStruct((N_GRAPHS, N_CLASSES), f32),
  )(p3, p1, sum2, sel2, pids2d, Wd, bd2d)

  return out
